# baseline (speedup 1.0000x reference)
import functools

import jax
import jax.numpy as jnp
from jax.experimental import pallas as pl
from jax.experimental.pallas import tpu as pltpu

N_BITS = 3
N_LEVELS = 2 ** N_BITS
B, H, S_NEW, D = 8, 16, 32, 64
S_MAX = 4096
BH = B * H
SP = S_MAX // 2
DP = 2 * D


def _calib_rows(x_full):
    xmin = jnp.min(x_full, axis=(0, 1))
    xmax = jnp.max(x_full, axis=(0, 1))
    scale = (xmax - xmin) / (N_LEVELS - 1)
    scale = jnp.maximum(scale, 1e-8)
    zp = jnp.round(-xmin / scale)
    scale2 = jnp.concatenate([scale, scale]).reshape(1, DP)
    zp2 = jnp.concatenate([zp, zp]).reshape(1, DP)
    return scale2, zp2


def _fused_kernel(pos_ref, k_ref, v_ref, kc_ref, vc_ref, out_ref, calib_ref):
    i = pl.program_id(0)

    @pl.when(i == 0)
    def _():
        kscale, kzp = _calib_rows(k_ref[:])
        vscale, vzp = _calib_rows(v_ref[:])
        calib_ref[0:1, :] = kscale
        calib_ref[1:2, :] = kzp
        calib_ref[2:3, :] = vscale
        calib_ref[3:4, :] = vzp

    kscale2 = calib_ref[0:1, :]
    kzp2 = calib_ref[1:2, :]
    vscale2 = calib_ref[2:3, :]
    vzp2 = calib_ref[3:4, :]

    out_ref[0, 0] = (kc_ref[0].astype(jnp.float32) - kzp2) * kscale2
    out_ref[1, 0] = (vc_ref[0].astype(jnp.float32) - vzp2) * vscale2

    kscale = kscale2[0:1, 0:D]
    kzp = kzp2[0:1, 0:D]
    vscale = vscale2[0:1, 0:D]
    vzp = vzp2[0:1, 0:D]
    kblk = k_ref[i]
    vblk = v_ref[i]
    kq = jnp.clip(jnp.round(kblk / kscale) + kzp, 0, N_LEVELS - 1)
    vq = jnp.clip(jnp.round(vblk / vscale) + vzp, 0, N_LEVELS - 1)
    knew = (kq - kzp) * kscale
    vnew = (vq - vzp) * vscale

    for j in range(S_NEW):
        p = pos_ref[j]
        r = p // 2
        even = (p % 2) == 0

        @pl.when(even)
        def _():
            out_ref[0, 0, pl.ds(r, 1), 0:D] = knew[j:j + 1, :]
            out_ref[1, 0, pl.ds(r, 1), 0:D] = vnew[j:j + 1, :]

        @pl.when(jnp.logical_not(even))
        def _():
            out_ref[0, 0, pl.ds(r, 1), D:DP] = knew[j:j + 1, :]
            out_ref[1, 0, pl.ds(r, 1), D:DP] = vnew[j:j + 1, :]


@functools.partial(jax.jit, static_argnames=("interpret",))
def kernel(k, v, positions, k_cache, v_cache, interpret=False):
    k_full = k.reshape(BH, S_NEW, D)
    v_full = v.reshape(BH, S_NEW, D)
    kc = k_cache.reshape(BH, SP, DP)
    vc = v_cache.reshape(BH, SP, DP)

    out = pl.pallas_call(
        _fused_kernel,
        grid=(BH,),
        in_specs=[
            pl.BlockSpec(memory_space=pltpu.SMEM),
            pl.BlockSpec((BH, S_NEW, D), lambda i: (0, 0, 0)),
            pl.BlockSpec((BH, S_NEW, D), lambda i: (0, 0, 0)),
            pl.BlockSpec((1, SP, DP), lambda i: (i, 0, 0)),
            pl.BlockSpec((1, SP, DP), lambda i: (i, 0, 0)),
        ],
        out_specs=pl.BlockSpec((2, 1, SP, DP), lambda i: (0, i, 0, 0)),
        out_shape=jax.ShapeDtypeStruct((2, BH, SP, DP), jnp.float32),
        scratch_shapes=[pltpu.VMEM((4, DP), jnp.float32)],
        interpret=interpret,
    )(positions, k_full, v_full, kc, vc)

    return out.reshape(2, B, H, S_MAX, D)


# speedup vs baseline: 1.3416x; 1.3416x over previous
import jax
import jax.numpy as jnp
from jax.experimental import pallas as pl
from jax.experimental.pallas import tpu as pltpu

N_BITS = 3
N_LEVELS = 2 ** N_BITS
B, H, S_NEW, D = 8, 16, 32, 64
S_MAX = 4096
BH = B * H


def _prologue_kernel(k_ref, v_ref, calib_ref, knew_ref, vnew_ref):
    def one(x, deq_ref, srow, zrow):
        flat = x.reshape(BH * S_NEW, D)
        xmin = jnp.min(flat, axis=0)
        xmax = jnp.max(flat, axis=0)
        scale = jnp.maximum((xmax - xmin) / (N_LEVELS - 1), 1e-8)
        zp = jnp.round(-xmin / scale)
        q = jnp.clip(jnp.round(flat / scale) + zp, 0, N_LEVELS - 1)
        deq_ref[:] = ((q - zp) * scale).reshape(B, H, S_NEW, D)
        calib_ref[srow:srow + 1, :] = scale.reshape(1, D)
        calib_ref[zrow:zrow + 1, :] = (-zp * scale).reshape(1, D)

    one(k_ref[:], knew_ref, 0, 1)
    one(v_ref[:], vnew_ref, 2, 3)


def _main_kernel(pos_ref, calib_ref, knew_ref, vnew_ref, kc_ref, vc_ref,
                 out_ref):
    kscale = calib_ref[0:1, :]
    kbias = calib_ref[1:2, :]
    vscale = calib_ref[2:3, :]
    vbias = calib_ref[3:4, :]

    out_ref[0, 0, 0] = kc_ref[0, 0].astype(jnp.float32) * kscale + kbias
    out_ref[1, 0, 0] = vc_ref[0, 0].astype(jnp.float32) * vscale + vbias

    for j in range(S_NEW):
        p = pos_ref[j]
        out_ref[0, 0, 0, pl.ds(p, 1), :] = knew_ref[0, 0, j:j + 1, :]
        out_ref[1, 0, 0, pl.ds(p, 1), :] = vnew_ref[0, 0, j:j + 1, :]


@jax.jit
def kernel(k, v, positions, k_cache, v_cache):
    calib, knew, vnew = pl.pallas_call(
        _prologue_kernel,
        out_shape=[
            jax.ShapeDtypeStruct((8, D), jnp.float32),
            jax.ShapeDtypeStruct((B, H, S_NEW, D), jnp.float32),
            jax.ShapeDtypeStruct((B, H, S_NEW, D), jnp.float32),
        ],
    )(k, v)

    out = pl.pallas_call(
        _main_kernel,
        grid=(BH,),
        in_specs=[
            pl.BlockSpec(memory_space=pltpu.SMEM),
            pl.BlockSpec((8, D), lambda i: (0, 0)),
            pl.BlockSpec((1, 1, S_NEW, D), lambda i: (i // H, i % H, 0, 0)),
            pl.BlockSpec((1, 1, S_NEW, D), lambda i: (i // H, i % H, 0, 0)),
            pl.BlockSpec((1, 1, S_MAX, D), lambda i: (i // H, i % H, 0, 0)),
            pl.BlockSpec((1, 1, S_MAX, D), lambda i: (i // H, i % H, 0, 0)),
        ],
        out_specs=pl.BlockSpec((2, 1, 1, S_MAX, D),
                               lambda i: (0, i // H, i % H, 0, 0)),
        out_shape=jax.ShapeDtypeStruct((2, B, H, S_MAX, D), jnp.float32),
    )(positions, calib, knew, vnew, k_cache, v_cache)

    return out
